# Initial kernel scaffold; baseline (speedup 1.0000x reference)
import jax
import jax.numpy as jnp
from jax.experimental import pallas as pl


def kernel(boxes, scores, ids, active_ids, dormant_ids):
    raise NotImplementedError("write your pallas kernel here")


# speedup vs baseline: 169.4394x; 169.4394x over previous
import functools

import jax
import jax.numpy as jnp
from jax import lax
from jax.experimental import pallas as pl
from jax.experimental.pallas import tpu as pltpu

_TRACK_THRESH = 0.3
_START_THRESH = 0.5
_RESUME_THRESH = 0.4
_NMS_THRESH = 0.5
_NEXT_ID = 1000

_N = 5000
_B = 128
_NB = 40
_NP = _NB * _B


def _boost_kernel(ids_ref, act_ref, boost_ref):
    ids = ids_ref[...]
    m = jnp.zeros(ids.shape, jnp.float32)

    def body(k, m):
        a = act_ref[0, k]
        return m + (ids == a).astype(jnp.float32)

    m = lax.fori_loop(0, act_ref.shape[1], body, m)
    boost_ref[...] = jnp.where((m > 0.0) & (ids >= 0), 1.0, 0.0)


def _nms_kernel(x1_ref, y1_ref, x2_ref, y2_ref,
                x1c_ref, y1c_ref, x2c_ref, y2c_ref, keep_ref):

    def iou_tile(j, bx1, by1, bx2, by2, barea):
        jx1 = x1c_ref[pl.ds(j, 1), :, :].reshape(_B, 1)
        jy1 = y1c_ref[pl.ds(j, 1), :, :].reshape(_B, 1)
        jx2 = x2c_ref[pl.ds(j, 1), :, :].reshape(_B, 1)
        jy2 = y2c_ref[pl.ds(j, 1), :, :].reshape(_B, 1)
        jarea = (jx2 - jx1) * (jy2 - jy1)
        xx1 = jnp.maximum(jx1, bx1)
        yy1 = jnp.maximum(jy1, by1)
        xx2 = jnp.minimum(jx2, bx2)
        yy2 = jnp.minimum(jy2, by2)
        w = jnp.maximum(xx2 - xx1, 0.0)
        h = jnp.maximum(yy2 - yy1, 0.0)
        inter = w * h
        iou = inter / (jarea + barea - inter + 1e-9)
        return (iou > _NMS_THRESH).astype(jnp.float32)

    def block_step(i, carry):
        bx1 = x1_ref[pl.ds(i, 1), :]
        by1 = y1_ref[pl.ds(i, 1), :]
        bx2 = x2_ref[pl.ds(i, 1), :]
        by2 = y2_ref[pl.ds(i, 1), :]
        barea = (bx2 - bx1) * (by2 - by1)

        def prev(j, sup):
            s = iou_tile(j, bx1, by1, bx2, by2, barea)
            kj = keep_ref[pl.ds(j, 1), :]
            return sup + jax.lax.dot(kj, s,
                                     preferred_element_type=jnp.float32)

        sup = lax.fori_loop(0, i, prev, jnp.zeros((1, _B), jnp.float32))
        base = (sup == 0.0).astype(jnp.float32)

        m = iou_tile(i, bx1, by1, bx2, by2, barea)
        subl = lax.broadcasted_iota(jnp.int32, (_B, _B), 0)
        lanel = lax.broadcasted_iota(jnp.int32, (_B, _B), 1)
        m = m * (subl < lanel).astype(jnp.float32)

        def in_cond(st):
            k_old, k = st
            return jnp.any(k_old != k)

        def in_body(st):
            _, k = st
            sup_in = jax.lax.dot(k, m, preferred_element_type=jnp.float32)
            k_new = base * (sup_in == 0.0).astype(jnp.float32)
            return k, k_new

        k0 = base
        k1 = base * (jax.lax.dot(k0, m,
                                 preferred_element_type=jnp.float32)
                     == 0.0).astype(jnp.float32)
        _, kf = lax.while_loop(in_cond, in_body, (k0, k1))
        keep_ref[pl.ds(i, 1), :] = kf
        return carry

    lax.fori_loop(0, _NB, block_step, jnp.zeros((1, 1), jnp.float32))


def _post_kernel(keep_ref, boosted_ref, ids_ref, dorm_ref,
                 fs_ref, fid_ref, keepo_ref, resume_ref):
    keep = keep_ref[...] > 0.5
    boosted = boosted_ref[...]
    ids = ids_ref[...]

    s = jnp.where(boosted >= 2.0, boosted - 2.0, boosted)
    s = jnp.where(s >= 1.0, s - 1.0, s)

    valid = lax.broadcasted_iota(jnp.int32, (_NB, _B), 0) * _B + \
        lax.broadcasted_iota(jnp.int32, (_NB, _B), 1) < _N
    start = keep & (ids < 0) & (s >= _START_THRESH) & valid
    startf = start.astype(jnp.float32)

    subl = lax.broadcasted_iota(jnp.int32, (_B, _B), 0)
    lanel = lax.broadcasted_iota(jnp.int32, (_B, _B), 1)
    upper = (subl <= lanel).astype(jnp.float32)
    rowcum = jax.lax.dot(startf, upper,
                         preferred_element_type=jnp.float32)
    totals = rowcum[:, _B - 1:_B]
    rsub = lax.broadcasted_iota(jnp.int32, (_NB, _NB), 0)
    rlan = lax.broadcasted_iota(jnp.int32, (_NB, _NB), 1)
    lower = (rlan < rsub).astype(jnp.float32)
    offs = jax.lax.dot(lower, totals,
                       preferred_element_type=jnp.float32)
    cum = rowcum + offs

    new_ids = (_NEXT_ID - 1 + cum).astype(jnp.int32)
    ids2 = jnp.where(start, new_ids, ids)

    inactive = keep & (ids2 >= 0) & (s < _TRACK_THRESH)

    dormf = jnp.zeros(ids.shape, jnp.float32)

    def body(k, m):
        a = dorm_ref[0, k]
        return m + (ids == a).astype(jnp.float32)

    dormf = lax.fori_loop(0, dorm_ref.shape[1], body, dormf)
    resume = keep & (dormf > 0.0) & (s >= _RESUME_THRESH)

    fs_ref[...] = s * keep.astype(jnp.float32)
    fid_ref[...] = jnp.where(inactive, -1, ids2)
    keepo_ref[...] = keep.astype(jnp.int32)
    resume_ref[...] = resume.astype(jnp.int32)


@jax.jit
def kernel(boxes, scores, ids, active_ids, dormant_ids):
    ids = ids.astype(jnp.int32)
    pad = _NP - _N
    ids_p = jnp.pad(ids, (0, pad), constant_values=-1).reshape(_NB, _B)
    scores_p = jnp.pad(scores, (0, pad)).reshape(_NB, _B)
    act = active_ids.astype(jnp.int32).reshape(1, -1)
    dorm = dormant_ids.astype(jnp.int32).reshape(1, -1)

    boost = pl.pallas_call(
        _boost_kernel,
        out_shape=jax.ShapeDtypeStruct((_NB, _B), jnp.float32),
        in_specs=[pl.BlockSpec(memory_space=pltpu.VMEM),
                  pl.BlockSpec(memory_space=pltpu.SMEM)],
        out_specs=pl.BlockSpec(memory_space=pltpu.VMEM),
    )(ids_p, act)

    boosted = scores_p + boost
    boosted_flat = boosted.reshape(_NP)
    sort_key = jnp.where(jnp.arange(_NP) < _N, boosted_flat, -jnp.inf)
    order = jnp.argsort(-sort_key, stable=True)

    boxes_p = jnp.pad(boxes, ((0, pad), (0, 0)))
    sb = boxes_p[order]
    sx1 = sb[:, 0].reshape(_NB, _B)
    sy1 = sb[:, 1].reshape(_NB, _B)
    sx2 = sb[:, 2].reshape(_NB, _B)
    sy2 = sb[:, 3].reshape(_NB, _B)

    keep_sorted = pl.pallas_call(
        _nms_kernel,
        out_shape=jax.ShapeDtypeStruct((_NB, _B), jnp.float32),
        in_specs=[pl.BlockSpec(memory_space=pltpu.VMEM)] * 8,
        out_specs=pl.BlockSpec(memory_space=pltpu.VMEM),
    )(sx1, sy1, sx2, sy2,
      sx1[:, :, None], sy1[:, :, None], sx2[:, :, None], sy2[:, :, None])

    inv = jnp.zeros((_NP,), jnp.int32).at[order].set(
        jnp.arange(_NP, dtype=jnp.int32))
    keep_orig = keep_sorted.reshape(_NP)[inv].reshape(_NB, _B)

    fs, fid, keepo, resume = pl.pallas_call(
        _post_kernel,
        out_shape=(jax.ShapeDtypeStruct((_NB, _B), jnp.float32),
                   jax.ShapeDtypeStruct((_NB, _B), jnp.int32),
                   jax.ShapeDtypeStruct((_NB, _B), jnp.int32),
                   jax.ShapeDtypeStruct((_NB, _B), jnp.int32)),
        in_specs=[pl.BlockSpec(memory_space=pltpu.VMEM)] * 3 +
                 [pl.BlockSpec(memory_space=pltpu.SMEM)],
        out_specs=(pl.BlockSpec(memory_space=pltpu.VMEM),) * 4,
    )(keep_orig, boosted, ids_p, dorm)

    fs = fs.reshape(_NP)[:_N]
    fid = fid.reshape(_NP)[:_N]
    keepo = keepo.reshape(_NP)[:_N] > 0
    resume = resume.reshape(_NP)[:_N] > 0
    return fs, fid, keepo, resume


# speedup vs baseline: 380.3563x; 2.2448x over previous
import functools

import jax
import jax.numpy as jnp
from jax import lax
from jax.experimental import pallas as pl
from jax.experimental.pallas import tpu as pltpu

_TRACK_THRESH = 0.3
_START_THRESH = 0.5
_RESUME_THRESH = 0.4
_NMS_THRESH = 0.5
_NEXT_ID = 1000

_N = 5000
_B = 128
_NB = 40
_NP = _NB * _B
_BS = 512
_NBS = _NP // _BS


def _boost_kernel(ids_ref, act_ref, boost_ref):
    ids = ids_ref[...]
    m = jnp.zeros(ids.shape, jnp.float32)

    def body(k, m):
        a = act_ref[0, k]
        return m + (ids == a).astype(jnp.float32)

    m = lax.fori_loop(0, act_ref.shape[1], body, m)
    boost_ref[...] = jnp.where((m > 0.0) & (ids >= 0), 1.0, 0.0)


def _nms_kernel(x1_ref, y1_ref, x2_ref, y2_ref,
                x1c_ref, y1c_ref, x2c_ref, y2c_ref, keep_ref):

    def iou_tile(j, bx1, by1, bx2, by2, barea):
        jx1 = x1c_ref[pl.ds(j, 1), :, :].reshape(_BS, 1)
        jy1 = y1c_ref[pl.ds(j, 1), :, :].reshape(_BS, 1)
        jx2 = x2c_ref[pl.ds(j, 1), :, :].reshape(_BS, 1)
        jy2 = y2c_ref[pl.ds(j, 1), :, :].reshape(_BS, 1)
        jarea = (jx2 - jx1) * (jy2 - jy1)
        xx1 = jnp.maximum(jx1, bx1)
        yy1 = jnp.maximum(jy1, by1)
        xx2 = jnp.minimum(jx2, bx2)
        yy2 = jnp.minimum(jy2, by2)
        w = jnp.maximum(xx2 - xx1, 0.0)
        h = jnp.maximum(yy2 - yy1, 0.0)
        inter = w * h
        iou = inter / (jarea + barea - inter + 1e-9)
        return (iou > _NMS_THRESH).astype(jnp.float32)

    def block_step(i, carry):
        bx1 = x1_ref[pl.ds(i, 1), :]
        by1 = y1_ref[pl.ds(i, 1), :]
        bx2 = x2_ref[pl.ds(i, 1), :]
        by2 = y2_ref[pl.ds(i, 1), :]
        barea = (bx2 - bx1) * (by2 - by1)

        def prev(j, sup):
            s = iou_tile(j, bx1, by1, bx2, by2, barea)
            kj = keep_ref[pl.ds(j, 1), :]
            return sup + jax.lax.dot(kj, s,
                                     preferred_element_type=jnp.float32)

        sup = lax.fori_loop(0, i, prev, jnp.zeros((1, _BS), jnp.float32))
        base = (sup == 0.0).astype(jnp.float32)

        m = iou_tile(i, bx1, by1, bx2, by2, barea)
        subl = lax.broadcasted_iota(jnp.int32, (_BS, _BS), 0)
        lanel = lax.broadcasted_iota(jnp.int32, (_BS, _BS), 1)
        m = m * (subl < lanel).astype(jnp.float32)

        def in_cond(st):
            k_old, k = st
            return jnp.any(k_old != k)

        def in_body(st):
            _, k = st
            sup_in = jax.lax.dot(k, m, preferred_element_type=jnp.float32)
            k_new = base * (sup_in == 0.0).astype(jnp.float32)
            return k, k_new

        k0 = base
        k1 = base * (jax.lax.dot(k0, m,
                                 preferred_element_type=jnp.float32)
                     == 0.0).astype(jnp.float32)
        _, kf = lax.while_loop(in_cond, in_body, (k0, k1))
        keep_ref[pl.ds(i, 1), :] = kf
        return carry

    lax.fori_loop(0, _NBS, block_step, jnp.zeros((1, 1), jnp.float32))


def _post_kernel(keep_ref, boosted_ref, ids_ref, dorm_ref,
                 fs_ref, fid_ref, keepo_ref, resume_ref):
    keep = keep_ref[...] > 0.5
    boosted = boosted_ref[...]
    ids = ids_ref[...]

    s = jnp.where(boosted >= 2.0, boosted - 2.0, boosted)
    s = jnp.where(s >= 1.0, s - 1.0, s)

    valid = lax.broadcasted_iota(jnp.int32, (_NB, _B), 0) * _B + \
        lax.broadcasted_iota(jnp.int32, (_NB, _B), 1) < _N
    start = keep & (ids < 0) & (s >= _START_THRESH) & valid
    startf = start.astype(jnp.float32)

    subl = lax.broadcasted_iota(jnp.int32, (_B, _B), 0)
    lanel = lax.broadcasted_iota(jnp.int32, (_B, _B), 1)
    upper = (subl <= lanel).astype(jnp.float32)
    rowcum = jax.lax.dot(startf, upper,
                         preferred_element_type=jnp.float32)
    totals = rowcum[:, _B - 1:_B]
    rsub = lax.broadcasted_iota(jnp.int32, (_NB, _NB), 0)
    rlan = lax.broadcasted_iota(jnp.int32, (_NB, _NB), 1)
    lower = (rlan < rsub).astype(jnp.float32)
    offs = jax.lax.dot(lower, totals,
                       preferred_element_type=jnp.float32)
    cum = rowcum + offs

    new_ids = (_NEXT_ID - 1 + cum).astype(jnp.int32)
    ids2 = jnp.where(start, new_ids, ids)

    inactive = keep & (ids2 >= 0) & (s < _TRACK_THRESH)

    dormf = jnp.zeros(ids.shape, jnp.float32)

    def body(k, m):
        a = dorm_ref[0, k]
        return m + (ids == a).astype(jnp.float32)

    dormf = lax.fori_loop(0, dorm_ref.shape[1], body, dormf)
    resume = keep & (dormf > 0.0) & (s >= _RESUME_THRESH)

    fs_ref[...] = s * keep.astype(jnp.float32)
    fid_ref[...] = jnp.where(inactive, -1, ids2)
    keepo_ref[...] = keep.astype(jnp.int32)
    resume_ref[...] = resume.astype(jnp.int32)


@jax.jit
def kernel(boxes, scores, ids, active_ids, dormant_ids):
    ids = ids.astype(jnp.int32)
    pad = _NP - _N
    ids_p = jnp.pad(ids, (0, pad), constant_values=-1).reshape(_NB, _B)
    scores_p = jnp.pad(scores, (0, pad)).reshape(_NB, _B)
    act = active_ids.astype(jnp.int32).reshape(1, -1)
    dorm = dormant_ids.astype(jnp.int32).reshape(1, -1)

    boost = pl.pallas_call(
        _boost_kernel,
        out_shape=jax.ShapeDtypeStruct((_NB, _B), jnp.float32),
        in_specs=[pl.BlockSpec(memory_space=pltpu.VMEM),
                  pl.BlockSpec(memory_space=pltpu.SMEM)],
        out_specs=pl.BlockSpec(memory_space=pltpu.VMEM),
    )(ids_p, act)

    boosted = scores_p + boost
    boosted_flat = boosted.reshape(_NP)
    sort_key = jnp.where(jnp.arange(_NP) < _N, boosted_flat, -jnp.inf)
    order = jnp.argsort(-sort_key, stable=True)

    boxes_p = jnp.pad(boxes, ((0, pad), (0, 0)))
    sb = boxes_p[order]
    sx1 = sb[:, 0].reshape(_NBS, _BS)
    sy1 = sb[:, 1].reshape(_NBS, _BS)
    sx2 = sb[:, 2].reshape(_NBS, _BS)
    sy2 = sb[:, 3].reshape(_NBS, _BS)

    keep_sorted = pl.pallas_call(
        _nms_kernel,
        out_shape=jax.ShapeDtypeStruct((_NBS, _BS), jnp.float32),
        in_specs=[pl.BlockSpec(memory_space=pltpu.VMEM)] * 8,
        out_specs=pl.BlockSpec(memory_space=pltpu.VMEM),
    )(sx1, sy1, sx2, sy2,
      sx1[:, :, None], sy1[:, :, None], sx2[:, :, None], sy2[:, :, None])

    inv = jnp.zeros((_NP,), jnp.int32).at[order].set(
        jnp.arange(_NP, dtype=jnp.int32))
    keep_orig = keep_sorted.reshape(_NP)[inv].reshape(_NB, _B)

    fs, fid, keepo, resume = pl.pallas_call(
        _post_kernel,
        out_shape=(jax.ShapeDtypeStruct((_NB, _B), jnp.float32),
                   jax.ShapeDtypeStruct((_NB, _B), jnp.int32),
                   jax.ShapeDtypeStruct((_NB, _B), jnp.int32),
                   jax.ShapeDtypeStruct((_NB, _B), jnp.int32)),
        in_specs=[pl.BlockSpec(memory_space=pltpu.VMEM)] * 3 +
                 [pl.BlockSpec(memory_space=pltpu.SMEM)],
        out_specs=(pl.BlockSpec(memory_space=pltpu.VMEM),) * 4,
    )(keep_orig, boosted, ids_p, dorm)

    fs = fs.reshape(_NP)[:_N]
    fid = fid.reshape(_NP)[:_N]
    keepo = keepo.reshape(_NP)[:_N] > 0
    resume = resume.reshape(_NP)[:_N] > 0
    return fs, fid, keepo, resume


# speedup vs baseline: 406.4030x; 1.0685x over previous
import functools

import jax
import jax.numpy as jnp
from jax import lax
from jax.experimental import pallas as pl
from jax.experimental.pallas import tpu as pltpu

_TRACK_THRESH = 0.3
_START_THRESH = 0.5
_RESUME_THRESH = 0.4
_NMS_THRESH = 0.5
_NEXT_ID = 1000

_N = 5000
_B = 128
_NB = 40
_NP = _NB * _B
_BS = 1024
_NBS = _NP // _BS


def _boost_kernel(ids_ref, act_ref, boost_ref):
    ids = ids_ref[...]
    m = jnp.zeros(ids.shape, jnp.float32)

    def body(k, m):
        a = act_ref[0, k]
        return m + (ids == a).astype(jnp.float32)

    m = lax.fori_loop(0, act_ref.shape[1], body, m)
    boost_ref[...] = jnp.where((m > 0.0) & (ids >= 0), 1.0, 0.0)


def _nms_kernel(x1_ref, y1_ref, x2_ref, y2_ref,
                x1c_ref, y1c_ref, x2c_ref, y2c_ref, keep_ref):

    def iou_tile(j, bx1, by1, bx2, by2, barea):
        jx1 = x1c_ref[pl.ds(j, 1), :, :].reshape(_BS, 1)
        jy1 = y1c_ref[pl.ds(j, 1), :, :].reshape(_BS, 1)
        jx2 = x2c_ref[pl.ds(j, 1), :, :].reshape(_BS, 1)
        jy2 = y2c_ref[pl.ds(j, 1), :, :].reshape(_BS, 1)
        jarea = (jx2 - jx1) * (jy2 - jy1)
        xx1 = jnp.maximum(jx1, bx1)
        yy1 = jnp.maximum(jy1, by1)
        xx2 = jnp.minimum(jx2, bx2)
        yy2 = jnp.minimum(jy2, by2)
        w = jnp.maximum(xx2 - xx1, 0.0)
        h = jnp.maximum(yy2 - yy1, 0.0)
        inter = w * h
        iou = inter / (jarea + barea - inter + 1e-9)
        return (iou > _NMS_THRESH).astype(jnp.float32)

    def block_step(i, carry):
        bx1 = x1_ref[pl.ds(i, 1), :]
        by1 = y1_ref[pl.ds(i, 1), :]
        bx2 = x2_ref[pl.ds(i, 1), :]
        by2 = y2_ref[pl.ds(i, 1), :]
        barea = (bx2 - bx1) * (by2 - by1)

        def prev(j, sup):
            s = iou_tile(j, bx1, by1, bx2, by2, barea)
            kj = keep_ref[pl.ds(j, 1), :]
            return sup + jax.lax.dot(kj, s,
                                     preferred_element_type=jnp.float32)

        sup = lax.fori_loop(0, i, prev, jnp.zeros((1, _BS), jnp.float32))
        base = (sup == 0.0).astype(jnp.float32)

        m = iou_tile(i, bx1, by1, bx2, by2, barea)
        subl = lax.broadcasted_iota(jnp.int32, (_BS, _BS), 0)
        lanel = lax.broadcasted_iota(jnp.int32, (_BS, _BS), 1)
        m = m * (subl < lanel).astype(jnp.float32)

        def in_cond(st):
            k_old, k = st
            return jnp.any(k_old != k)

        def in_body(st):
            _, k = st
            sup_in = jax.lax.dot(k, m, preferred_element_type=jnp.float32)
            k_new = base * (sup_in == 0.0).astype(jnp.float32)
            return k, k_new

        k0 = base
        k1 = base * (jax.lax.dot(k0, m,
                                 preferred_element_type=jnp.float32)
                     == 0.0).astype(jnp.float32)
        _, kf = lax.while_loop(in_cond, in_body, (k0, k1))
        keep_ref[pl.ds(i, 1), :] = kf
        return carry

    lax.fori_loop(0, _NBS, block_step, jnp.zeros((1, 1), jnp.float32))


def _post_kernel(keep_ref, boosted_ref, ids_ref, dorm_ref,
                 fs_ref, fid_ref, keepo_ref, resume_ref):
    keep = keep_ref[...] > 0.5
    boosted = boosted_ref[...]
    ids = ids_ref[...]

    s = jnp.where(boosted >= 2.0, boosted - 2.0, boosted)
    s = jnp.where(s >= 1.0, s - 1.0, s)

    valid = lax.broadcasted_iota(jnp.int32, (_NB, _B), 0) * _B + \
        lax.broadcasted_iota(jnp.int32, (_NB, _B), 1) < _N
    start = keep & (ids < 0) & (s >= _START_THRESH) & valid
    startf = start.astype(jnp.float32)

    subl = lax.broadcasted_iota(jnp.int32, (_B, _B), 0)
    lanel = lax.broadcasted_iota(jnp.int32, (_B, _B), 1)
    upper = (subl <= lanel).astype(jnp.float32)
    rowcum = jax.lax.dot(startf, upper,
                         preferred_element_type=jnp.float32)
    totals = rowcum[:, _B - 1:_B]
    rsub = lax.broadcasted_iota(jnp.int32, (_NB, _NB), 0)
    rlan = lax.broadcasted_iota(jnp.int32, (_NB, _NB), 1)
    lower = (rlan < rsub).astype(jnp.float32)
    offs = jax.lax.dot(lower, totals,
                       preferred_element_type=jnp.float32)
    cum = rowcum + offs

    new_ids = (_NEXT_ID - 1 + cum).astype(jnp.int32)
    ids2 = jnp.where(start, new_ids, ids)

    inactive = keep & (ids2 >= 0) & (s < _TRACK_THRESH)

    dormf = jnp.zeros(ids.shape, jnp.float32)

    def body(k, m):
        a = dorm_ref[0, k]
        return m + (ids == a).astype(jnp.float32)

    dormf = lax.fori_loop(0, dorm_ref.shape[1], body, dormf)
    resume = keep & (dormf > 0.0) & (s >= _RESUME_THRESH)

    fs_ref[...] = s * keep.astype(jnp.float32)
    fid_ref[...] = jnp.where(inactive, -1, ids2)
    keepo_ref[...] = keep.astype(jnp.int32)
    resume_ref[...] = resume.astype(jnp.int32)


@jax.jit
def kernel(boxes, scores, ids, active_ids, dormant_ids):
    ids = ids.astype(jnp.int32)
    pad = _NP - _N
    ids_p = jnp.pad(ids, (0, pad), constant_values=-1).reshape(_NB, _B)
    scores_p = jnp.pad(scores, (0, pad)).reshape(_NB, _B)
    act = active_ids.astype(jnp.int32).reshape(1, -1)
    dorm = dormant_ids.astype(jnp.int32).reshape(1, -1)

    boost = pl.pallas_call(
        _boost_kernel,
        out_shape=jax.ShapeDtypeStruct((_NB, _B), jnp.float32),
        in_specs=[pl.BlockSpec(memory_space=pltpu.VMEM),
                  pl.BlockSpec(memory_space=pltpu.SMEM)],
        out_specs=pl.BlockSpec(memory_space=pltpu.VMEM),
    )(ids_p, act)

    boosted = scores_p + boost
    boosted_flat = boosted.reshape(_NP)
    sort_key = jnp.where(jnp.arange(_NP) < _N, boosted_flat, -jnp.inf)
    order = jnp.argsort(-sort_key, stable=True)

    boxes_p = jnp.pad(boxes, ((0, pad), (0, 0)))
    sb = boxes_p[order]
    sx1 = sb[:, 0].reshape(_NBS, _BS)
    sy1 = sb[:, 1].reshape(_NBS, _BS)
    sx2 = sb[:, 2].reshape(_NBS, _BS)
    sy2 = sb[:, 3].reshape(_NBS, _BS)

    keep_sorted = pl.pallas_call(
        _nms_kernel,
        out_shape=jax.ShapeDtypeStruct((_NBS, _BS), jnp.float32),
        in_specs=[pl.BlockSpec(memory_space=pltpu.VMEM)] * 8,
        out_specs=pl.BlockSpec(memory_space=pltpu.VMEM),
    )(sx1, sy1, sx2, sy2,
      sx1[:, :, None], sy1[:, :, None], sx2[:, :, None], sy2[:, :, None])

    inv = jnp.zeros((_NP,), jnp.int32).at[order].set(
        jnp.arange(_NP, dtype=jnp.int32))
    keep_orig = keep_sorted.reshape(_NP)[inv].reshape(_NB, _B)

    fs, fid, keepo, resume = pl.pallas_call(
        _post_kernel,
        out_shape=(jax.ShapeDtypeStruct((_NB, _B), jnp.float32),
                   jax.ShapeDtypeStruct((_NB, _B), jnp.int32),
                   jax.ShapeDtypeStruct((_NB, _B), jnp.int32),
                   jax.ShapeDtypeStruct((_NB, _B), jnp.int32)),
        in_specs=[pl.BlockSpec(memory_space=pltpu.VMEM)] * 3 +
                 [pl.BlockSpec(memory_space=pltpu.SMEM)],
        out_specs=(pl.BlockSpec(memory_space=pltpu.VMEM),) * 4,
    )(keep_orig, boosted, ids_p, dorm)

    fs = fs.reshape(_NP)[:_N]
    fid = fid.reshape(_NP)[:_N]
    keepo = keepo.reshape(_NP)[:_N] > 0
    resume = resume.reshape(_NP)[:_N] > 0
    return fs, fid, keepo, resume


# speedup vs baseline: 444.6089x; 1.0940x over previous
import functools

import jax
import jax.numpy as jnp
from jax import lax
from jax.experimental import pallas as pl
from jax.experimental.pallas import tpu as pltpu
from jax.experimental.pallas import tpu_sc as plsc

_TRACK_THRESH = 0.3
_START_THRESH = 0.5
_RESUME_THRESH = 0.4
_NMS_THRESH = 0.5
_NEXT_ID = 1000

_N = 5000
_B = 128
_NB = 40
_NP = _NB * _B
_BS = 1024
_NBS = _NP // _BS


_SC_NW = 32
_SC_D = 128
_GPW = _NP // _SC_NW
_GCH = 80

_sc_mesh = plsc.VectorSubcoreMesh(core_axis_name="c", subcore_axis_name="s")


@functools.partial(
    pl.kernel, mesh=_sc_mesh,
    out_type=jax.ShapeDtypeStruct((_NP, _SC_D), jnp.float32),
    scratch_types=[pltpu.VMEM((_GCH,), jnp.int32),
                   pltpu.VMEM((_GCH, _SC_D), jnp.float32),
                   pltpu.SemaphoreType.DMA],
)
def _sc_gather(table_hbm, idx_hbm, out_hbm, idx_v, rows_v, sem):
    wid = lax.axis_index("s") * 2 + lax.axis_index("c")
    for c in range(_GPW // _GCH):
        base = wid * _GPW + c * _GCH
        pltpu.sync_copy(idx_hbm.at[pl.ds(base, _GCH)], idx_v)
        pltpu.async_copy(table_hbm.at[idx_v], rows_v, sem).wait()
        pltpu.sync_copy(rows_v, out_hbm.at[pl.ds(base, _GCH)])


@functools.partial(
    pl.kernel, mesh=_sc_mesh,
    out_type=jax.ShapeDtypeStruct((_NP, _SC_D), jnp.float32),
    scratch_types=[pltpu.VMEM((_GCH,), jnp.int32),
                   pltpu.VMEM((_GCH, _SC_D), jnp.float32),
                   pltpu.SemaphoreType.DMA],
)
def _sc_scatter(rows_hbm, idx_hbm, out_hbm, idx_v, rows_v, sem):
    wid = lax.axis_index("s") * 2 + lax.axis_index("c")
    for c in range(_GPW // _GCH):
        base = wid * _GPW + c * _GCH
        pltpu.sync_copy(idx_hbm.at[pl.ds(base, _GCH)], idx_v)
        pltpu.sync_copy(rows_hbm.at[pl.ds(base, _GCH)], rows_v)
        pltpu.async_copy(rows_v, out_hbm.at[idx_v], sem).wait()


def _boost_kernel(ids_ref, act_ref, boost_ref):
    ids = ids_ref[...]
    m = jnp.zeros(ids.shape, jnp.float32)

    def body(k, m):
        a = act_ref[0, k]
        return m + (ids == a).astype(jnp.float32)

    m = lax.fori_loop(0, act_ref.shape[1], body, m)
    boost_ref[...] = jnp.where((m > 0.0) & (ids >= 0), 1.0, 0.0)


def _nms_kernel(x1_ref, y1_ref, x2_ref, y2_ref,
                x1c_ref, y1c_ref, x2c_ref, y2c_ref, keep_ref):

    def iou_tile(j, bx1, by1, bx2, by2, barea):
        jx1 = x1c_ref[pl.ds(j, 1), :, :].reshape(_BS, 1)
        jy1 = y1c_ref[pl.ds(j, 1), :, :].reshape(_BS, 1)
        jx2 = x2c_ref[pl.ds(j, 1), :, :].reshape(_BS, 1)
        jy2 = y2c_ref[pl.ds(j, 1), :, :].reshape(_BS, 1)
        jarea = (jx2 - jx1) * (jy2 - jy1)
        xx1 = jnp.maximum(jx1, bx1)
        yy1 = jnp.maximum(jy1, by1)
        xx2 = jnp.minimum(jx2, bx2)
        yy2 = jnp.minimum(jy2, by2)
        w = jnp.maximum(xx2 - xx1, 0.0)
        h = jnp.maximum(yy2 - yy1, 0.0)
        inter = w * h
        iou = inter / (jarea + barea - inter + 1e-9)
        return (iou > _NMS_THRESH).astype(jnp.float32)

    def block_step(i, carry):
        bx1 = x1_ref[pl.ds(i, 1), :]
        by1 = y1_ref[pl.ds(i, 1), :]
        bx2 = x2_ref[pl.ds(i, 1), :]
        by2 = y2_ref[pl.ds(i, 1), :]
        barea = (bx2 - bx1) * (by2 - by1)

        def prev(j, sup):
            s = iou_tile(j, bx1, by1, bx2, by2, barea)
            kj = keep_ref[pl.ds(j, 1), :]
            return sup + jax.lax.dot(kj, s,
                                     preferred_element_type=jnp.float32)

        sup = lax.fori_loop(0, i, prev, jnp.zeros((1, _BS), jnp.float32))
        base = (sup == 0.0).astype(jnp.float32)

        m = iou_tile(i, bx1, by1, bx2, by2, barea)
        subl = lax.broadcasted_iota(jnp.int32, (_BS, _BS), 0)
        lanel = lax.broadcasted_iota(jnp.int32, (_BS, _BS), 1)
        m = m * (subl < lanel).astype(jnp.float32)

        def in_cond(st):
            k_old, k = st
            return jnp.any(k_old != k)

        def in_body(st):
            _, k = st
            sup_in = jax.lax.dot(k, m, preferred_element_type=jnp.float32)
            k_new = base * (sup_in == 0.0).astype(jnp.float32)
            return k, k_new

        k0 = base
        k1 = base * (jax.lax.dot(k0, m,
                                 preferred_element_type=jnp.float32)
                     == 0.0).astype(jnp.float32)
        _, kf = lax.while_loop(in_cond, in_body, (k0, k1))
        keep_ref[pl.ds(i, 1), :] = kf
        return carry

    lax.fori_loop(0, _NBS, block_step, jnp.zeros((1, 1), jnp.float32))


def _post_kernel(keep_ref, boosted_ref, ids_ref, dorm_ref,
                 fs_ref, fid_ref, keepo_ref, resume_ref):
    keep = keep_ref[...] > 0.5
    boosted = boosted_ref[...]
    ids = ids_ref[...]

    s = jnp.where(boosted >= 2.0, boosted - 2.0, boosted)
    s = jnp.where(s >= 1.0, s - 1.0, s)

    valid = lax.broadcasted_iota(jnp.int32, (_NB, _B), 0) * _B + \
        lax.broadcasted_iota(jnp.int32, (_NB, _B), 1) < _N
    start = keep & (ids < 0) & (s >= _START_THRESH) & valid
    startf = start.astype(jnp.float32)

    subl = lax.broadcasted_iota(jnp.int32, (_B, _B), 0)
    lanel = lax.broadcasted_iota(jnp.int32, (_B, _B), 1)
    upper = (subl <= lanel).astype(jnp.float32)
    rowcum = jax.lax.dot(startf, upper,
                         preferred_element_type=jnp.float32)
    totals = rowcum[:, _B - 1:_B]
    rsub = lax.broadcasted_iota(jnp.int32, (_NB, _NB), 0)
    rlan = lax.broadcasted_iota(jnp.int32, (_NB, _NB), 1)
    lower = (rlan < rsub).astype(jnp.float32)
    offs = jax.lax.dot(lower, totals,
                       preferred_element_type=jnp.float32)
    cum = rowcum + offs

    new_ids = (_NEXT_ID - 1 + cum).astype(jnp.int32)
    ids2 = jnp.where(start, new_ids, ids)

    inactive = keep & (ids2 >= 0) & (s < _TRACK_THRESH)

    dormf = jnp.zeros(ids.shape, jnp.float32)

    def body(k, m):
        a = dorm_ref[0, k]
        return m + (ids == a).astype(jnp.float32)

    dormf = lax.fori_loop(0, dorm_ref.shape[1], body, dormf)
    resume = keep & (dormf > 0.0) & (s >= _RESUME_THRESH)

    fs_ref[...] = s * keep.astype(jnp.float32)
    fid_ref[...] = jnp.where(inactive, -1, ids2)
    keepo_ref[...] = keep.astype(jnp.int32)
    resume_ref[...] = resume.astype(jnp.int32)


@jax.jit
def kernel(boxes, scores, ids, active_ids, dormant_ids):
    ids = ids.astype(jnp.int32)
    pad = _NP - _N
    ids_p = jnp.pad(ids, (0, pad), constant_values=-1).reshape(_NB, _B)
    scores_p = jnp.pad(scores, (0, pad)).reshape(_NB, _B)
    act = active_ids.astype(jnp.int32).reshape(1, -1)
    dorm = dormant_ids.astype(jnp.int32).reshape(1, -1)

    boost = pl.pallas_call(
        _boost_kernel,
        out_shape=jax.ShapeDtypeStruct((_NB, _B), jnp.float32),
        in_specs=[pl.BlockSpec(memory_space=pltpu.VMEM),
                  pl.BlockSpec(memory_space=pltpu.SMEM)],
        out_specs=pl.BlockSpec(memory_space=pltpu.VMEM),
    )(ids_p, act)

    boosted = scores_p + boost
    boosted_flat = boosted.reshape(_NP)
    sort_key = jnp.where(jnp.arange(_NP) < _N, boosted_flat, -jnp.inf)
    order = jnp.argsort(-sort_key, stable=True)

    boxes_p = jnp.pad(boxes, ((0, pad), (0, 0)))
    boxes16 = jnp.pad(boxes_p, ((0, 0), (0, _SC_D - 4)))
    sb = _sc_gather(boxes16, order)
    sx1 = sb[:, 0].reshape(_NBS, _BS)
    sy1 = sb[:, 1].reshape(_NBS, _BS)
    sx2 = sb[:, 2].reshape(_NBS, _BS)
    sy2 = sb[:, 3].reshape(_NBS, _BS)

    keep_sorted = pl.pallas_call(
        _nms_kernel,
        out_shape=jax.ShapeDtypeStruct((_NBS, _BS), jnp.float32),
        in_specs=[pl.BlockSpec(memory_space=pltpu.VMEM)] * 8,
        out_specs=pl.BlockSpec(memory_space=pltpu.VMEM),
    )(sx1, sy1, sx2, sy2,
      sx1[:, :, None], sy1[:, :, None], sx2[:, :, None], sy2[:, :, None])

    keep16 = jnp.broadcast_to(keep_sorted.reshape(_NP, 1), (_NP, _SC_D))
    keep_scat = _sc_scatter(keep16, order)
    keep_orig = keep_scat[:, 0].reshape(_NB, _B)

    fs, fid, keepo, resume = pl.pallas_call(
        _post_kernel,
        out_shape=(jax.ShapeDtypeStruct((_NB, _B), jnp.float32),
                   jax.ShapeDtypeStruct((_NB, _B), jnp.int32),
                   jax.ShapeDtypeStruct((_NB, _B), jnp.int32),
                   jax.ShapeDtypeStruct((_NB, _B), jnp.int32)),
        in_specs=[pl.BlockSpec(memory_space=pltpu.VMEM)] * 3 +
                 [pl.BlockSpec(memory_space=pltpu.SMEM)],
        out_specs=(pl.BlockSpec(memory_space=pltpu.VMEM),) * 4,
    )(keep_orig, boosted, ids_p, dorm)

    fs = fs.reshape(_NP)[:_N]
    fid = fid.reshape(_NP)[:_N]
    keepo = keepo.reshape(_NP)[:_N] > 0
    resume = resume.reshape(_NP)[:_N] > 0
    return fs, fid, keepo, resume


# speedup vs baseline: 455.2652x; 1.0240x over previous
import functools

import jax
import jax.numpy as jnp
from jax import lax
from jax.experimental import pallas as pl
from jax.experimental.pallas import tpu as pltpu
from jax.experimental.pallas import tpu_sc as plsc

_TRACK_THRESH = 0.3
_START_THRESH = 0.5
_RESUME_THRESH = 0.4
_NMS_THRESH = 0.5
_NEXT_ID = 1000

_N = 5000
_B = 128
_NB = 40
_NP = _NB * _B
_BS = 1024
_NBS = _NP // _BS


_SC_NW = 32
_SC_D = 128
_GPW = _NP // _SC_NW
_GCH = 80

_sc_mesh = plsc.VectorSubcoreMesh(core_axis_name="c", subcore_axis_name="s")


@functools.partial(
    pl.kernel, mesh=_sc_mesh,
    out_type=jax.ShapeDtypeStruct((_NP, _SC_D), jnp.float32),
    scratch_types=[pltpu.VMEM((_GCH,), jnp.int32),
                   pltpu.VMEM((_GCH, _SC_D), jnp.float32),
                   pltpu.SemaphoreType.DMA],
)
def _sc_gather(table_hbm, idx_hbm, out_hbm, idx_v, rows_v, sem):
    wid = lax.axis_index("s") * 2 + lax.axis_index("c")
    for c in range(_GPW // _GCH):
        base = wid * _GPW + c * _GCH
        pltpu.sync_copy(idx_hbm.at[pl.ds(base, _GCH)], idx_v)
        pltpu.async_copy(table_hbm.at[idx_v], rows_v, sem).wait()
        pltpu.sync_copy(rows_v, out_hbm.at[pl.ds(base, _GCH)])


@functools.partial(
    pl.kernel, mesh=_sc_mesh,
    out_type=jax.ShapeDtypeStruct((_NP, _SC_D), jnp.float32),
    scratch_types=[pltpu.VMEM((_GCH,), jnp.int32),
                   pltpu.VMEM((_GCH, _SC_D), jnp.float32),
                   pltpu.SemaphoreType.DMA],
)
def _sc_scatter(rows_hbm, idx_hbm, out_hbm, idx_v, rows_v, sem):
    wid = lax.axis_index("s") * 2 + lax.axis_index("c")
    for c in range(_GPW // _GCH):
        base = wid * _GPW + c * _GCH
        pltpu.sync_copy(idx_hbm.at[pl.ds(base, _GCH)], idx_v)
        pltpu.sync_copy(rows_hbm.at[pl.ds(base, _GCH)], rows_v)
        pltpu.async_copy(rows_v, out_hbm.at[idx_v], sem).wait()


def _boost_kernel(ids_ref, act_ref, boost_ref):
    ids = ids_ref[...]
    act = act_ref[...]
    m = (ids[None, :, :] == act).astype(jnp.float32).sum(axis=0)
    boost_ref[...] = jnp.where((m > 0.0) & (ids >= 0), 1.0, 0.0)


def _nms_kernel(x1_ref, y1_ref, x2_ref, y2_ref,
                x1c_ref, y1c_ref, x2c_ref, y2c_ref, keep_ref):

    def iou_tile(j, bx1, by1, bx2, by2, barea):
        jx1 = x1c_ref[pl.ds(j, 1), :, :].reshape(_BS, 1)
        jy1 = y1c_ref[pl.ds(j, 1), :, :].reshape(_BS, 1)
        jx2 = x2c_ref[pl.ds(j, 1), :, :].reshape(_BS, 1)
        jy2 = y2c_ref[pl.ds(j, 1), :, :].reshape(_BS, 1)
        jarea = (jx2 - jx1) * (jy2 - jy1)
        xx1 = jnp.maximum(jx1, bx1)
        yy1 = jnp.maximum(jy1, by1)
        xx2 = jnp.minimum(jx2, bx2)
        yy2 = jnp.minimum(jy2, by2)
        w = jnp.maximum(xx2 - xx1, 0.0)
        h = jnp.maximum(yy2 - yy1, 0.0)
        inter = w * h
        iou = inter / (jarea + barea - inter + 1e-9)
        return (iou > _NMS_THRESH).astype(jnp.float32)

    def block_step(i, carry):
        bx1 = x1_ref[pl.ds(i, 1), :]
        by1 = y1_ref[pl.ds(i, 1), :]
        bx2 = x2_ref[pl.ds(i, 1), :]
        by2 = y2_ref[pl.ds(i, 1), :]
        barea = (bx2 - bx1) * (by2 - by1)

        def prev(j, sup):
            s = iou_tile(j, bx1, by1, bx2, by2, barea)
            kj = keep_ref[pl.ds(j, 1), :]
            return sup + jax.lax.dot(kj, s,
                                     preferred_element_type=jnp.float32)

        sup = lax.fori_loop(0, i, prev, jnp.zeros((1, _BS), jnp.float32))
        base = (sup == 0.0).astype(jnp.float32)

        m = iou_tile(i, bx1, by1, bx2, by2, barea)
        subl = lax.broadcasted_iota(jnp.int32, (_BS, _BS), 0)
        lanel = lax.broadcasted_iota(jnp.int32, (_BS, _BS), 1)
        m = m * (subl < lanel).astype(jnp.float32)

        def in_cond(st):
            k_old, k = st
            return jnp.any(k_old != k)

        def in_body(st):
            _, k = st
            sup_in = jax.lax.dot(k, m, preferred_element_type=jnp.float32)
            k_new = base * (sup_in == 0.0).astype(jnp.float32)
            return k, k_new

        k0 = base
        k1 = base * (jax.lax.dot(k0, m,
                                 preferred_element_type=jnp.float32)
                     == 0.0).astype(jnp.float32)
        _, kf = lax.while_loop(in_cond, in_body, (k0, k1))
        keep_ref[pl.ds(i, 1), :] = kf
        return carry

    lax.fori_loop(0, _NBS, block_step, jnp.zeros((1, 1), jnp.float32))


def _post_kernel(keep_ref, boosted_ref, ids_ref, dorm_ref,
                 fs_ref, fid_ref, keepo_ref, resume_ref):
    keep = keep_ref[...] > 0.5
    boosted = boosted_ref[...]
    ids = ids_ref[...]

    s = jnp.where(boosted >= 2.0, boosted - 2.0, boosted)
    s = jnp.where(s >= 1.0, s - 1.0, s)

    valid = lax.broadcasted_iota(jnp.int32, (_NB, _B), 0) * _B + \
        lax.broadcasted_iota(jnp.int32, (_NB, _B), 1) < _N
    start = keep & (ids < 0) & (s >= _START_THRESH) & valid
    startf = start.astype(jnp.float32)

    subl = lax.broadcasted_iota(jnp.int32, (_B, _B), 0)
    lanel = lax.broadcasted_iota(jnp.int32, (_B, _B), 1)
    upper = (subl <= lanel).astype(jnp.float32)
    rowcum = jax.lax.dot(startf, upper,
                         preferred_element_type=jnp.float32)
    totals = rowcum[:, _B - 1:_B]
    rsub = lax.broadcasted_iota(jnp.int32, (_NB, _NB), 0)
    rlan = lax.broadcasted_iota(jnp.int32, (_NB, _NB), 1)
    lower = (rlan < rsub).astype(jnp.float32)
    offs = jax.lax.dot(lower, totals,
                       preferred_element_type=jnp.float32)
    cum = rowcum + offs

    new_ids = (_NEXT_ID - 1 + cum).astype(jnp.int32)
    ids2 = jnp.where(start, new_ids, ids)

    inactive = keep & (ids2 >= 0) & (s < _TRACK_THRESH)

    dormf = (ids[None, :, :] == dorm_ref[...]).astype(jnp.float32).sum(axis=0)
    resume = keep & (dormf > 0.0) & (s >= _RESUME_THRESH)

    fs_ref[...] = s * keep.astype(jnp.float32)
    fid_ref[...] = jnp.where(inactive, -1, ids2)
    keepo_ref[...] = keep.astype(jnp.int32)
    resume_ref[...] = resume.astype(jnp.int32)


@jax.jit
def kernel(boxes, scores, ids, active_ids, dormant_ids):
    ids = ids.astype(jnp.int32)
    pad = _NP - _N
    ids_p = jnp.pad(ids, (0, pad), constant_values=-1).reshape(_NB, _B)
    scores_p = jnp.pad(scores, (0, pad)).reshape(_NB, _B)
    act = active_ids.astype(jnp.int32).reshape(-1, 1, 1)
    dorm = dormant_ids.astype(jnp.int32).reshape(-1, 1, 1)

    boost = pl.pallas_call(
        _boost_kernel,
        out_shape=jax.ShapeDtypeStruct((_NB, _B), jnp.float32),
        in_specs=[pl.BlockSpec(memory_space=pltpu.VMEM),
                  pl.BlockSpec(memory_space=pltpu.VMEM)],
        out_specs=pl.BlockSpec(memory_space=pltpu.VMEM),
    )(ids_p, act)

    boosted = scores_p + boost
    boosted_flat = boosted.reshape(_NP)
    sort_key = jnp.where(jnp.arange(_NP) < _N, boosted_flat, -jnp.inf)
    order = jnp.argsort(-sort_key, stable=True)

    boxes_p = jnp.pad(boxes, ((0, pad), (0, 0)))
    boxes16 = jnp.pad(boxes_p, ((0, 0), (0, _SC_D - 4)))
    sb = _sc_gather(boxes16, order)
    sx1 = sb[:, 0].reshape(_NBS, _BS)
    sy1 = sb[:, 1].reshape(_NBS, _BS)
    sx2 = sb[:, 2].reshape(_NBS, _BS)
    sy2 = sb[:, 3].reshape(_NBS, _BS)

    keep_sorted = pl.pallas_call(
        _nms_kernel,
        out_shape=jax.ShapeDtypeStruct((_NBS, _BS), jnp.float32),
        in_specs=[pl.BlockSpec(memory_space=pltpu.VMEM)] * 8,
        out_specs=pl.BlockSpec(memory_space=pltpu.VMEM),
    )(sx1, sy1, sx2, sy2,
      sx1[:, :, None], sy1[:, :, None], sx2[:, :, None], sy2[:, :, None])

    keep16 = jnp.broadcast_to(keep_sorted.reshape(_NP, 1), (_NP, _SC_D))
    keep_scat = _sc_scatter(keep16, order)
    keep_orig = keep_scat[:, 0].reshape(_NB, _B)

    fs, fid, keepo, resume = pl.pallas_call(
        _post_kernel,
        out_shape=(jax.ShapeDtypeStruct((_NB, _B), jnp.float32),
                   jax.ShapeDtypeStruct((_NB, _B), jnp.int32),
                   jax.ShapeDtypeStruct((_NB, _B), jnp.int32),
                   jax.ShapeDtypeStruct((_NB, _B), jnp.int32)),
        in_specs=[pl.BlockSpec(memory_space=pltpu.VMEM)] * 4,
        out_specs=(pl.BlockSpec(memory_space=pltpu.VMEM),) * 4,
    )(keep_orig, boosted, ids_p, dorm)

    fs = fs.reshape(_NP)[:_N]
    fid = fid.reshape(_NP)[:_N]
    keepo = keepo.reshape(_NP)[:_N] > 0
    resume = resume.reshape(_NP)[:_N] > 0
    return fs, fid, keepo, resume
